# baseline (speedup 1.0000x reference)
import jax
import jax.numpy as jnp
import numpy as np
from jax import lax
from jax.experimental import pallas as pl
from jax.experimental.pallas import tpu as pltpu
from jax.experimental.pallas import tpu_sc as plsc

_ANCHORS = [0.28, 0.22, 0.38, 0.48, 0.90, 0.78]
_NCLS = 80
_IGN = 0.5

_NB, _NA, _NR, _NC = 16, 3, 64, 64
_CELLS = _NR * _NC
_MASK_N = _NA * _CELLS
_TPB = 112
_NCHUNK = _TPB // 16

_AW = [float(np.float32(_ANCHORS[2 * a]) * np.float32(64.0)) for a in range(3)]
_AH = [float(np.float32(_ANCHORS[2 * a + 1]) * np.float32(64.0)) for a in range(3)]
_ADEN = [
    float(np.float32(np.float32(_AW[a]) * np.float32(_AH[a])) + np.float32(1e-16))
    for a in range(3)
]


def _sc_body(tgt_hbm, noobj_hbm, obj_hbm, tgt_v, obj_v, noobj_v):
    c = lax.axis_index("c")
    s = lax.axis_index("s")
    wid = s * 2 + c

    @pl.when(wid < _NB)
    def _():
        b = wid
        zi = jnp.zeros((16,), jnp.int32)
        oi = jnp.ones((16,), jnp.int32)

        def mbody(i, carry):
            base = i * 128
            for j in range(8):
                obj_v[pl.ds(base + j * 16, 16)] = zi
                noobj_v[pl.ds(base + j * 16, 16)] = oi
            return carry
        lax.fori_loop(0, _MASK_N // 128, mbody, 0)

        pltpu.sync_copy(tgt_hbm.at[pl.ds(b * (6 * _TPB), 6 * _TPB)], tgt_v)

        lane = lax.iota(jnp.int32, 16)

        def cbody(k, carry):
            off = k * 16
            tx = tgt_v[pl.ds(2 * _TPB + off, 16)]
            ty = tgt_v[pl.ds(3 * _TPB + off, 16)]
            tw = tgt_v[pl.ds(4 * _TPB + off, 16)]
            th = tgt_v[pl.ds(5 * _TPB + off, 16)]
            gi = (tx * 64.0).astype(jnp.int32)
            gj = (ty * 64.0).astype(jnp.int32)
            gw = tw * 64.0
            gh = th * 64.0
            area = gw * gh
            ious = []
            for a in range(3):
                inter = jnp.minimum(_AW[a], gw) * jnp.minimum(_AH[a], gh)
                ious.append(inter / (_ADEN[a] + area - inter))
            best_iou = ious[0]
            best_n = zi
            m1 = ious[1] > best_iou
            best_iou = jnp.where(m1, ious[1], best_iou)
            best_n = jnp.where(m1, 1, best_n)
            m2 = ious[2] > best_iou
            best_n = jnp.where(m2, 2, best_n)

            valid = (off + lane) < 100
            cell = gi * 64 + gj
            plsc.store_scatter(obj_v, [best_n * _CELLS + cell], oi, mask=valid)
            for a in range(3):
                kill = valid & ((best_n == a) | (ious[a] > _IGN))
                plsc.store_scatter(noobj_v, [a * _CELLS + cell], zi, mask=kill)
            return carry
        lax.fori_loop(0, _NCHUNK, cbody, 0)

        boff_m = b * _MASK_N
        pltpu.sync_copy(obj_v, obj_hbm.at[pl.ds(boff_m, _MASK_N)])
        pltpu.sync_copy(noobj_v, noobj_hbm.at[pl.ds(boff_m, _MASK_N)])


def _tc_body(tgt_ref, out_ref):
    tgt = tgt_ref[0]
    tx = tgt[2:3, :]
    ty = tgt[3:4, :]
    tw = tgt[4:5, :]
    th = tgt[5:6, :]
    gi = (tx * 64.0).astype(jnp.int32)
    gj = (ty * 64.0).astype(jnp.int32)
    gw = tw * 64.0
    gh = th * 64.0
    area = gw * gh
    ious = []
    for a in range(3):
        inter = jnp.minimum(_AW[a], gw) * jnp.minimum(_AH[a], gh)
        ious.append(inter / (_ADEN[a] + area - inter))
    best_iou = ious[0]
    best_n = jnp.zeros((1, _TPB), jnp.int32)
    m1 = ious[1] > best_iou
    best_iou = jnp.where(m1, ious[1], best_iou)
    best_n = jnp.where(m1, 1, best_n)
    m2 = ious[2] > best_iou
    best_n = jnp.where(m2, 2, best_n)

    tvalid = lax.broadcasted_iota(jnp.int32, (1, _TPB), 1) < 100
    midx = jnp.where(tvalid, best_n * _CELLS + gi * 64 + gj, -1)
    cls_ = tgt[1:2, :].astype(jnp.int32)

    cell_iota = lax.broadcasted_iota(jnp.int32, (_MASK_N, _TPB), 0)
    at_mat = jnp.where(cell_iota == midx, 1.0, 0.0).astype(jnp.bfloat16)
    cls_iota = lax.broadcasted_iota(jnp.int32, (_NCLS, _TPB), 0)
    b_mat = jnp.where(cls_iota == cls_, 1.0, 0.0).astype(jnp.bfloat16)
    cnt = jax.lax.dot_general(
        at_mat, b_mat, (((1,), (1,)), ((), ())),
        preferred_element_type=jnp.float32)
    out_ref[...] = jnp.minimum(cnt, 1.0).reshape(1, _NA, _NR, _NC, _NCLS)


@jax.jit
def _run(tgt3):
    sc_fn = pl.kernel(
        _sc_body,
        out_type=(
            jax.ShapeDtypeStruct((_NB * _MASK_N,), jnp.int32),
            jax.ShapeDtypeStruct((_NB * _MASK_N,), jnp.int32),
        ),
        mesh=plsc.VectorSubcoreMesh(core_axis_name="c", subcore_axis_name="s"),
        compiler_params=pltpu.CompilerParams(
            needs_layout_passes=False, skip_device_barrier=True),
        scratch_types=[
            pltpu.VMEM((6 * _TPB,), jnp.float32),
            pltpu.VMEM((_MASK_N,), jnp.int32),
            pltpu.VMEM((_MASK_N,), jnp.int32),
        ],
    )
    tcls = pl.pallas_call(
        _tc_body,
        grid=(_NB,),
        in_specs=[pl.BlockSpec((1, 6, _TPB), lambda b: (b, 0, 0))],
        out_specs=pl.BlockSpec(
            (1, _NA, _NR, _NC, _NCLS), lambda b: (b, 0, 0, 0, 0)),
        out_shape=jax.ShapeDtypeStruct(
            (_NB, _NA, _NR, _NC, _NCLS), jnp.float32),
    )(tgt3)
    noobj_f, obj_f = sc_fn(tgt3.reshape(-1))
    return noobj_f, obj_f, tcls


def kernel(x, target):
    nT = target.shape[0]
    tgt3 = target.reshape(nT // _NB, _NB, 6).transpose(1, 2, 0)
    tgt3 = jnp.pad(tgt3, ((0, 0), (0, 0), (0, _TPB - nT // _NB)))
    noobj_f, obj_f, tcls = _run(tgt3)
    return (
        noobj_f.reshape(_NB, _NA, _NR, _NC),
        obj_f.reshape(_NB, _NA, _NR, _NC),
        tcls,
    )


# speedup vs baseline: 1.0034x; 1.0034x over previous
import jax
import jax.numpy as jnp
import numpy as np
from jax import lax
from jax.experimental import pallas as pl
from jax.experimental.pallas import tpu as pltpu
from jax.experimental.pallas import tpu_sc as plsc

_ANCHORS = [0.28, 0.22, 0.38, 0.48, 0.90, 0.78]
_NCLS = 80
_IGN = 0.5

_NB, _NA, _NR, _NC = 16, 3, 64, 64
_CELLS = _NR * _NC
_MASK_N = _NA * _CELLS
_TPB = 112
_NCHUNK = _TPB // 16

_AW = [float(np.float32(_ANCHORS[2 * a]) * np.float32(64.0)) for a in range(3)]
_AH = [float(np.float32(_ANCHORS[2 * a + 1]) * np.float32(64.0)) for a in range(3)]
_ADEN = [
    float(np.float32(np.float32(_AW[a]) * np.float32(_AH[a])) + np.float32(1e-16))
    for a in range(3)
]


def _sc_body(tgt_hbm, noobj_hbm, obj_hbm, tgt_v, obj_v, noobj_v):
    c = lax.axis_index("c")
    s = lax.axis_index("s")
    wid = s * 2 + c

    @pl.when(wid < _NB)
    def _():
        b = wid
        zi = jnp.zeros((16,), jnp.int32)
        oi = jnp.ones((16,), jnp.int32)

        def mbody(i, carry):
            base = i * 128
            for j in range(8):
                obj_v[pl.ds(base + j * 16, 16)] = zi
                noobj_v[pl.ds(base + j * 16, 16)] = oi
            return carry
        lax.fori_loop(0, _MASK_N // 128, mbody, 0)

        pltpu.sync_copy(tgt_hbm.at[pl.ds(b * (6 * _TPB), 6 * _TPB)], tgt_v)

        lane = lax.iota(jnp.int32, 16)

        def cbody(k, carry):
            off = k * 16
            tx = tgt_v[pl.ds(2 * _TPB + off, 16)]
            ty = tgt_v[pl.ds(3 * _TPB + off, 16)]
            tw = tgt_v[pl.ds(4 * _TPB + off, 16)]
            th = tgt_v[pl.ds(5 * _TPB + off, 16)]
            gi = (tx * 64.0).astype(jnp.int32)
            gj = (ty * 64.0).astype(jnp.int32)
            gw = tw * 64.0
            gh = th * 64.0
            area = gw * gh
            ious = []
            for a in range(3):
                inter = jnp.minimum(_AW[a], gw) * jnp.minimum(_AH[a], gh)
                ious.append(inter / (_ADEN[a] + area - inter))
            best_iou = ious[0]
            best_n = zi
            m1 = ious[1] > best_iou
            best_iou = jnp.where(m1, ious[1], best_iou)
            best_n = jnp.where(m1, 1, best_n)
            m2 = ious[2] > best_iou
            best_n = jnp.where(m2, 2, best_n)

            valid = (off + lane) < 100
            cell = gi * 64 + gj
            plsc.store_scatter(obj_v, [best_n * _CELLS + cell], oi, mask=valid)
            for a in range(3):
                kill = valid & ((best_n == a) | (ious[a] > _IGN))
                plsc.store_scatter(noobj_v, [a * _CELLS + cell], zi, mask=kill)
            return carry
        lax.fori_loop(0, _NCHUNK, cbody, 0)

        boff_m = b * _MASK_N
        pltpu.sync_copy(obj_v, obj_hbm.at[pl.ds(boff_m, _MASK_N)])
        pltpu.sync_copy(noobj_v, noobj_hbm.at[pl.ds(boff_m, _MASK_N)])


def _tc_body(tgt_ref, out_ref):
    tgt = tgt_ref[0]
    tx = tgt[2:3, :]
    ty = tgt[3:4, :]
    tw = tgt[4:5, :]
    th = tgt[5:6, :]
    gi = (tx * 64.0).astype(jnp.int32)
    gj = (ty * 64.0).astype(jnp.int32)
    gw = tw * 64.0
    gh = th * 64.0
    area = gw * gh
    ious = []
    for a in range(3):
        inter = jnp.minimum(_AW[a], gw) * jnp.minimum(_AH[a], gh)
        ious.append(inter / (_ADEN[a] + area - inter))
    best_iou = ious[0]
    best_n = jnp.zeros((1, _TPB), jnp.int32)
    m1 = ious[1] > best_iou
    best_iou = jnp.where(m1, ious[1], best_iou)
    best_n = jnp.where(m1, 1, best_n)
    m2 = ious[2] > best_iou
    best_n = jnp.where(m2, 2, best_n)

    tvalid = lax.broadcasted_iota(jnp.int32, (1, _TPB), 1) < 100
    midx = jnp.where(tvalid, best_n * _CELLS + gi * 64 + gj, -1)
    cls_ = tgt[1:2, :].astype(jnp.int32)

    cell_iota = lax.broadcasted_iota(jnp.int32, (_MASK_N, _TPB), 0)
    at_mat = jnp.where(cell_iota == midx, 1.0, 0.0).astype(jnp.bfloat16)
    cls_iota = lax.broadcasted_iota(jnp.int32, (_NCLS, _TPB), 0)
    b_mat = jnp.where(cls_iota == cls_, 1.0, 0.0).astype(jnp.bfloat16)
    cnt = jax.lax.dot_general(
        at_mat, b_mat, (((1,), (1,)), ((), ())),
        preferred_element_type=jnp.float32)
    out_ref[...] = jnp.minimum(cnt, 1.0).reshape(1, _NA, _NR, _NC, _NCLS)


@jax.jit
def _run(tgt3):
    sc_fn = pl.kernel(
        _sc_body,
        out_type=(
            jax.ShapeDtypeStruct((_NB * _MASK_N,), jnp.int32),
            jax.ShapeDtypeStruct((_NB * _MASK_N,), jnp.int32),
        ),
        mesh=plsc.VectorSubcoreMesh(core_axis_name="c", subcore_axis_name="s"),
        compiler_params=pltpu.CompilerParams(needs_layout_passes=False),
        scratch_types=[
            pltpu.VMEM((6 * _TPB,), jnp.float32),
            pltpu.VMEM((_MASK_N,), jnp.int32),
            pltpu.VMEM((_MASK_N,), jnp.int32),
        ],
    )
    tcls = pl.pallas_call(
        _tc_body,
        grid=(_NB,),
        in_specs=[pl.BlockSpec((1, 6, _TPB), lambda b: (b, 0, 0))],
        out_specs=pl.BlockSpec(
            (1, _NA, _NR, _NC, _NCLS), lambda b: (b, 0, 0, 0, 0)),
        out_shape=jax.ShapeDtypeStruct(
            (_NB, _NA, _NR, _NC, _NCLS), jnp.float32),
    )(tgt3)
    noobj_f, obj_f = sc_fn(tgt3.reshape(-1))
    return noobj_f, obj_f, tcls


def kernel(x, target):
    nT = target.shape[0]
    tgt3 = target.reshape(nT // _NB, _NB, 6).transpose(1, 2, 0)
    tgt3 = jnp.pad(tgt3, ((0, 0), (0, 0), (0, _TPB - nT // _NB)))
    noobj_f, obj_f, tcls = _run(tgt3)
    return (
        noobj_f.reshape(_NB, _NA, _NR, _NC),
        obj_f.reshape(_NB, _NA, _NR, _NC),
        tcls,
    )
